# baseline (speedup 1.0000x reference)
import jax
import jax.numpy as jnp
from jax import lax
from jax.experimental import pallas as pl
from jax.experimental.pallas import tpu as pltpu
from jax.experimental.pallas import tpu_sc as plsc

C = 6
N = 100000
H = 64
B = 16384
L = 16

NC, NS = 2, 16
NW = NC * NS
BW = B // NW
K = 128
NCH = BW // K
NJ = NCH * C
NG = K // L


def _sc_body(ub, ib, pf, qf, cvec, dpad, r_out, p_out,
             uix_v, iix_v, ubv, ibv, pb0, qb0, pb1, qb1, cv_v, dv_v,
             racc, pacc, semp0, semq0, semp1, semq1):
    wid = lax.axis_index("s") * NC + lax.axis_index("c")
    base = wid * BW

    pltpu.sync_copy(ub.at[pl.ds(base, BW)], ubv)
    pltpu.sync_copy(ib.at[pl.ds(base, BW)], ibv)
    pltpu.sync_copy(cvec, cv_v)
    pltpu.sync_copy(dpad, dv_v)

    zeros = jnp.zeros((L,), jnp.float32)
    for t in range(NCH):
        for g in range(NG):
            off = t * K + g * L
            uvec = ubv[pl.ds(off, L)]
            ivec = ibv[pl.ds(off, L)]
            for comp in range(C):
                j = t * C + comp
                uix_v[j, pl.ds(g * L, L)] = uvec + comp * N
                iix_v[j, pl.ds(g * L, L)] = ivec + comp * N
            racc[pl.ds(off, L)] = zeros
            pacc[pl.ds(off, L)] = zeros

    def fire(j, pb, qb, semp, semq):
        pltpu.async_copy(pf.at[uix_v.at[j]], pb, semp)
        pltpu.async_copy(qf.at[iix_v.at[j]], qb, semq)

    def wait(pb, qb, semp, semq):
        pltpu.make_async_copy(pf.at[uix_v.at[0]], pb, semp).wait()
        pltpu.make_async_copy(qf.at[iix_v.at[0]], qb, semq).wait()

    iota = lax.iota(jnp.int32, L)
    rowvs = [g * L + iota for g in range(NG)]

    def compute(j, pb, qb):
        t = j // C
        comp = lax.rem(j, C)
        cbase = comp * H
        dval = plsc.load_gather(dv_v, [jnp.full((L,), 0, jnp.int32) + comp])

        def h_body(h, accs):
            colv = jnp.full((L,), 0, jnp.int32) + h
            cv3 = plsc.load_gather(cv_v, [colv + cbase])
            out = []
            for g in range(NG):
                pv = plsc.load_gather(pb, [rowvs[g], colv])
                qv = plsc.load_gather(qb, [rowvs[g], colv])
                out.append(accs[2 * g] + pv * qv)
                out.append(accs[2 * g + 1] + qv * cv3)
            return tuple(out)

        accs = lax.fori_loop(0, H, h_body, (zeros,) * (2 * NG))
        for g in range(NG):
            off = t * K + g * L
            racc[pl.ds(off, L)] = racc[pl.ds(off, L)] + accs[2 * g]
            sig = 1.0 / (1.0 + jnp.exp(-(accs[2 * g + 1] + dval)))
            pacc[pl.ds(off, L)] = pacc[pl.ds(off, L)] + sig

    fire(0, pb0, qb0, semp0, semq0)

    def jj_body(jj, carry):
        j0 = 2 * jj
        j1 = j0 + 1
        fire(j1, pb1, qb1, semp1, semq1)
        wait(pb0, qb0, semp0, semq0)
        compute(j0, pb0, qb0)

        @pl.when(j1 + 1 < NJ)
        def _():
            fire(j1 + 1, pb0, qb0, semp0, semq0)

        wait(pb1, qb1, semp1, semq1)
        compute(j1, pb1, qb1)
        return carry

    lax.fori_loop(0, NJ // 2, jj_body, 0)

    inv_c = jnp.float32(1.0 / C)
    for z in range(BW // L):
        racc[pl.ds(z * L, L)] = racc[pl.ds(z * L, L)] * inv_c
        pacc[pl.ds(z * L, L)] = pacc[pl.ds(z * L, L)] * inv_c
    pltpu.sync_copy(racc, r_out.at[pl.ds(base, BW)])
    pltpu.sync_copy(pacc, p_out.at[pl.ds(base, BW)])


@jax.jit
def _sc_call(ub, ib, pf, qf, cvec, dpad):
    mesh = plsc.VectorSubcoreMesh(core_axis_name="c", subcore_axis_name="s")
    f = pl.kernel(
        _sc_body,
        out_type=[
            jax.ShapeDtypeStruct((B,), jnp.float32),
            jax.ShapeDtypeStruct((B,), jnp.float32),
        ],
        mesh=mesh,
        compiler_params=pltpu.CompilerParams(
            needs_layout_passes=False, use_tc_tiling_on_sc=False
        ),
        scratch_types=[
            pltpu.VMEM((NJ, K), jnp.int32),
            pltpu.VMEM((NJ, K), jnp.int32),
            pltpu.VMEM((BW,), jnp.int32),
            pltpu.VMEM((BW,), jnp.int32),
            pltpu.VMEM((K, H), jnp.float32),
            pltpu.VMEM((K, H), jnp.float32),
            pltpu.VMEM((K, H), jnp.float32),
            pltpu.VMEM((K, H), jnp.float32),
            pltpu.VMEM((C * H,), jnp.float32),
            pltpu.VMEM((L,), jnp.float32),
            pltpu.VMEM((BW,), jnp.float32),
            pltpu.VMEM((BW,), jnp.float32),
            pltpu.SemaphoreType.DMA,
            pltpu.SemaphoreType.DMA,
            pltpu.SemaphoreType.DMA,
            pltpu.SemaphoreType.DMA,
        ],
    )
    return f(ub, ib, pf, qf, cvec, dpad)


def kernel(u_batch, i_batch, P, Q, c, d):
    pf = P.reshape(C * N, H)
    qf = Q.reshape(C * N, H)
    cvec = c.reshape(C * H)
    dpad = jnp.concatenate([d.reshape(C), jnp.zeros((L - C,), jnp.float32)])
    r, p = _sc_call(u_batch, i_batch, pf, qf, cvec, dpad)
    return (r.reshape(B, 1), p.reshape(B, 1))
